# Initial kernel scaffold; baseline (speedup 1.0000x reference)
import jax
import jax.numpy as jnp
from jax.experimental import pallas as pl


def kernel(x_nchw, weight):
    raise NotImplementedError("write your pallas kernel here")


# speedup vs baseline: 1.0638x; 1.0638x over previous
import functools

import jax
import jax.numpy as jnp
from jax.experimental import pallas as pl
from jax.experimental.pallas import tpu as pltpu


def _shift_lanes(p, off):
    bb, c = p.shape
    z = jnp.zeros((bb, abs(off)), jnp.float32)
    if off == 0:
        return p
    if off > 0:
        return jnp.concatenate([z, p[:, : c - off]], axis=1)
    return jnp.concatenate([p[:, -off:], z], axis=1)


def _eca_block_kernel(x_ref, w_ref, o_ref, *, k):
    xb = x_ref[...]
    pooled = jnp.sum(xb.astype(jnp.float32), axis=-1)
    pad = (k - 1) // 2
    z = jnp.zeros_like(pooled)
    for j in range(k):
        z = z + w_ref[j : j + 1, :] * _shift_lanes(pooled, pad - j)
    gate = jax.nn.sigmoid(z)
    o_ref[...] = xb * gate.astype(xb.dtype)[:, :, None]


def kernel(x_nchw, weight):
    b, c, h, w = x_nchw.shape
    hw = h * w
    k = weight.shape[-1]
    x_flat = x_nchw.reshape(b, c, hw)
    w_kc = weight.reshape(c, k).T.astype(jnp.float32) / jnp.float32(hw)

    itemsize = jnp.dtype(x_nchw.dtype).itemsize
    budget = 48 * 1024 * 1024
    bb = 1
    for cand in (8, 4, 2):
        if b % cand == 0 and 4 * cand * c * hw * itemsize <= budget:
            bb = cand
            break
    block_bytes = bb * c * hw * itemsize
    limit = int(min(4 * block_bytes + (4 << 20), 56 << 20))

    out = pl.pallas_call(
        functools.partial(_eca_block_kernel, k=k),
        out_shape=jax.ShapeDtypeStruct((b, c, hw), x_nchw.dtype),
        grid=(b // bb,),
        in_specs=[
            pl.BlockSpec((bb, c, hw), lambda i: (i, 0, 0)),
            pl.BlockSpec((k, c), lambda i: (0, 0)),
        ],
        out_specs=pl.BlockSpec((bb, c, hw), lambda i: (i, 0, 0)),
        compiler_params=pltpu.CompilerParams(
            dimension_semantics=("parallel",),
            vmem_limit_bytes=limit,
        ),
    )(x_flat, w_kc)
    return out.reshape(b, c, h, w)
